# baseline (speedup 1.0000x reference)
import functools

import jax
import jax.numpy as jnp
from jax import lax
from jax.experimental import pallas as pl
from jax.experimental.pallas import tpu as pltpu
from jax.experimental.pallas import tpu_sc as plsc

_NC = 2
_NS = 16
_NW = _NC * _NS
_BB = 128
_BPT0 = 32
_BPT1 = 48


def _matmul_body(x_ref, w_ref, y_ref):
    y_ref[0] = jnp.dot(x_ref[...], w_ref[0], preferred_element_type=jnp.float32)


def _combine2_body(p_ref, q_ref, b_ref, o_ref):
    o_ref[...] = (p_ref[0] + p_ref[1]) + (q_ref[0] + q_ref[1]) + b_ref[...]


def _make_sc_scatter(n_acc, cout, bpt):
    rpt = n_acc // _NS

    mesh = plsc.VectorSubcoreMesh(
        core_axis_name="c", subcore_axis_name="s",
        num_cores=_NC, num_subcores=_NS)

    @functools.partial(
        pl.kernel,
        out_type=jax.ShapeDtypeStruct((_NC, n_acc, cout), jnp.float32),
        mesh=mesh,
        scratch_types=[
            pltpu.VMEM((bpt, _BB), jnp.int32),
            pltpu.VMEM((bpt, _BB), jnp.int32),
            pltpu.VMEM((2, _BB, cout), jnp.float32),
            pltpu.VMEM_SHARED((n_acc, cout), jnp.float32),
            pltpu.SemaphoreType.DMA,
            pltpu.SemaphoreType.DMA,
        ],
    )
    def sc_scatter(y_hbm, gidx_hbm, omap_hbm, out_hbm,
                   gidx_v, omap_v, rows_v, acc_sh, gs0, gs1):
        cid = lax.axis_index("c")
        sid = lax.axis_index("s")
        wid = cid * _NS + sid

        gsems = (gs0, gs1)

        def gather(j, b):
            return pltpu.async_copy(y_hbm.at[gidx_v.at[j]], rows_v.at[b],
                                    gsems[b])

        def gather_wait(j, b):
            pltpu.make_async_copy(y_hbm.at[gidx_v.at[j]], rows_v.at[b],
                                  gsems[b]).wait()

        def scatter_sync(j, b):
            pltpu.sync_copy(rows_v.at[b], acc_sh.at[omap_v.at[j]], add=True)

        pltpu.sync_copy(gidx_hbm.at[pl.ds(wid * bpt, bpt)], gidx_v)
        pltpu.sync_copy(omap_hbm.at[pl.ds(wid * bpt, bpt)], omap_v)
        gather(1, 1)

        zero = jnp.zeros((16,), jnp.float32)

        @pl.loop(0, _BB)
        def _zero_rows(r):
            for q in range(cout // 16):
                rows_v[0, r, pl.ds(q * 16, 16)] = zero

        for t in range(rpt // _BB):
            pltpu.sync_copy(rows_v.at[0],
                            acc_sh.at[pl.ds(sid * rpt + t * _BB, _BB)])
        gather(0, 0)
        plsc.subcore_barrier()

        @pl.loop(0, bpt // 2)
        def _pairs(jj):
            j0 = jj * 2
            j1 = j0 + 1
            gather_wait(j0, 0)
            scatter_sync(j0, 0)

            @pl.when(jj < bpt // 2 - 1)
            def _():
                gather(j0 + 2, 0)

            gather_wait(j1, 1)
            scatter_sync(j1, 1)

            @pl.when(jj < bpt // 2 - 1)
            def _():
                gather(j1 + 2, 1)

        plsc.subcore_barrier()
        pltpu.sync_copy(acc_sh.at[pl.ds(sid * rpt, rpt)],
                        out_hbm.at[cid, pl.ds(sid * rpt, rpt)])

    return sc_scatter


def _matmul(x, w, bm):
    n, cin = x.shape
    nk, _, cout = w.shape
    y = pl.pallas_call(
        _matmul_body,
        grid=(n // bm, nk),
        in_specs=[
            pl.BlockSpec((bm, cin), lambda i, k: (i, 0)),
            pl.BlockSpec((1, cin, cout), lambda i, k: (k, 0, 0)),
        ],
        out_specs=pl.BlockSpec((1, bm, cout), lambda i, k: (k, i, 0)),
        out_shape=jax.ShapeDtypeStruct((nk, n, cout), jnp.float32),
    )(x, w)
    return y.reshape(nk * n, cout)


def kernel(x, imap, omap, kernel, bias):
    n, cin = x.shape
    kvol, epk = imap.shape
    cout = kernel.shape[2]

    e_total = kvol * epk
    e0 = _NW * _BPT0 * _BB
    e1 = _NW * _BPT1 * _BB
    n_acc = ((n + _NS * _BB - 1) // (_NS * _BB)) * (_NS * _BB)

    k0 = -(-e0 // epk)
    k1 = e0 // epk

    gidx = (jnp.arange(kvol, dtype=jnp.int32)[:, None] * n + imap).ravel()
    omap_f = omap.ravel()
    pad = e0 + e1 - e_total
    pad_ids = jnp.arange(pad, dtype=jnp.int32)
    gidx = jnp.concatenate([gidx, k1 * n + (pad_ids * 53) % ((kvol - k1) * n)])
    omap_f = jnp.concatenate([omap_f, n + pad_ids % (n_acc - n)])

    gidx0 = gidx[:e0].reshape(e0 // _BB, _BB)
    omap0 = omap_f[:e0].reshape(e0 // _BB, _BB)
    gidx1 = (gidx[e0:] - k1 * n).reshape(e1 // _BB, _BB)
    omap1 = omap_f[e0:].reshape(e1 // _BB, _BB)

    bm = 2000
    y0 = _matmul(x, kernel[:k0], bm)
    y1 = _matmul(x, kernel[k1:], bm)
    p0 = _make_sc_scatter(n_acc, cout, _BPT0)(y0, gidx0, omap0)
    p1 = _make_sc_scatter(n_acc, cout, _BPT1)(y1, gidx1, omap1)

    br = 1000
    out = pl.pallas_call(
        _combine2_body,
        grid=(n // br,),
        in_specs=[
            pl.BlockSpec((_NC, br, cout), lambda i: (0, i, 0)),
            pl.BlockSpec((_NC, br, cout), lambda i: (0, i, 0)),
            pl.BlockSpec((1, cout), lambda i: (0, 0)),
        ],
        out_specs=pl.BlockSpec((br, cout), lambda i: (i, 0)),
        out_shape=jax.ShapeDtypeStruct((n, cout), jnp.float32),
    )(p0, p1, bias.reshape(1, cout))
    return out


# speedup vs baseline: 1.0287x; 1.0287x over previous
import functools

import jax
import jax.numpy as jnp
from jax import lax
from jax.experimental import pallas as pl
from jax.experimental.pallas import tpu as pltpu
from jax.experimental.pallas import tpu_sc as plsc

_NC = 2
_NS = 16
_NW = _NC * _NS
_BB = 128
_BPT0 = 40
_BPT1 = 40


def _matmul_body(x_ref, w_ref, y_ref):
    y_ref[0] = jnp.dot(x_ref[...], w_ref[0], preferred_element_type=jnp.float32)


def _combine2_body(p_ref, q_ref, b_ref, o_ref):
    o_ref[...] = (p_ref[0] + p_ref[1]) + (q_ref[0] + q_ref[1]) + b_ref[...]


def _make_sc_scatter(n_acc, cout, bpt):
    rpt = n_acc // _NS

    mesh = plsc.VectorSubcoreMesh(
        core_axis_name="c", subcore_axis_name="s",
        num_cores=_NC, num_subcores=_NS)

    @functools.partial(
        pl.kernel,
        out_type=jax.ShapeDtypeStruct((_NC, n_acc, cout), jnp.float32),
        mesh=mesh,
        scratch_types=[
            pltpu.VMEM((bpt, _BB), jnp.int32),
            pltpu.VMEM((bpt, _BB), jnp.int32),
            pltpu.VMEM((2, _BB, cout), jnp.float32),
            pltpu.VMEM_SHARED((n_acc, cout), jnp.float32),
            pltpu.SemaphoreType.DMA,
            pltpu.SemaphoreType.DMA,
        ],
    )
    def sc_scatter(y_hbm, gidx_hbm, omap_hbm, out_hbm,
                   gidx_v, omap_v, rows_v, acc_sh, gs0, gs1):
        cid = lax.axis_index("c")
        sid = lax.axis_index("s")
        wid = cid * _NS + sid

        gsems = (gs0, gs1)

        def gather(j, b):
            return pltpu.async_copy(y_hbm.at[gidx_v.at[j]], rows_v.at[b],
                                    gsems[b])

        def gather_wait(j, b):
            pltpu.make_async_copy(y_hbm.at[gidx_v.at[j]], rows_v.at[b],
                                  gsems[b]).wait()

        def scatter_sync(j, b):
            pltpu.sync_copy(rows_v.at[b], acc_sh.at[omap_v.at[j]], add=True)

        pltpu.sync_copy(gidx_hbm.at[pl.ds(wid * bpt, bpt)], gidx_v)
        pltpu.sync_copy(omap_hbm.at[pl.ds(wid * bpt, bpt)], omap_v)
        gather(1, 1)

        zero = jnp.zeros((16,), jnp.float32)

        @pl.loop(0, _BB)
        def _zero_rows(r):
            for q in range(cout // 16):
                rows_v[0, r, pl.ds(q * 16, 16)] = zero

        for t in range(rpt // _BB):
            pltpu.sync_copy(rows_v.at[0],
                            acc_sh.at[pl.ds(sid * rpt + t * _BB, _BB)])
        gather(0, 0)
        plsc.subcore_barrier()

        @pl.loop(0, bpt // 2)
        def _pairs(jj):
            j0 = jj * 2
            j1 = j0 + 1
            gather_wait(j0, 0)
            scatter_sync(j0, 0)

            @pl.when(jj < bpt // 2 - 1)
            def _():
                gather(j0 + 2, 0)

            gather_wait(j1, 1)
            scatter_sync(j1, 1)

            @pl.when(jj < bpt // 2 - 1)
            def _():
                gather(j1 + 2, 1)

        plsc.subcore_barrier()
        pltpu.sync_copy(acc_sh.at[pl.ds(sid * rpt, rpt)],
                        out_hbm.at[cid, pl.ds(sid * rpt, rpt)])

    return sc_scatter


def _matmul(x, w, bm):
    n, cin = x.shape
    nk, _, cout = w.shape
    y = pl.pallas_call(
        _matmul_body,
        grid=(n // bm, nk),
        in_specs=[
            pl.BlockSpec((bm, cin), lambda i, k: (i, 0)),
            pl.BlockSpec((1, cin, cout), lambda i, k: (k, 0, 0)),
        ],
        out_specs=pl.BlockSpec((1, bm, cout), lambda i, k: (k, i, 0)),
        out_shape=jax.ShapeDtypeStruct((nk, n, cout), jnp.float32),
    )(x, w)
    return y.reshape(nk * n, cout)


def kernel(x, imap, omap, kernel, bias):
    n, cin = x.shape
    kvol, epk = imap.shape
    cout = kernel.shape[2]

    e_total = kvol * epk
    e0 = _NW * _BPT0 * _BB
    e1 = _NW * _BPT1 * _BB
    n_acc = ((n + _NS * _BB - 1) // (_NS * _BB)) * (_NS * _BB)

    k0 = -(-e0 // epk)
    k1 = e0 // epk

    gidx = (jnp.arange(kvol, dtype=jnp.int32)[:, None] * n + imap).ravel()
    omap_f = omap.ravel()
    pad = e0 + e1 - e_total
    pad_ids = jnp.arange(pad, dtype=jnp.int32)
    gidx = jnp.concatenate([gidx, k1 * n + (pad_ids * 53) % ((kvol - k1) * n)])
    omap_f = jnp.concatenate([omap_f, n + pad_ids % (n_acc - n)])

    gidx0 = gidx[:e0].reshape(e0 // _BB, _BB)
    omap0 = omap_f[:e0].reshape(e0 // _BB, _BB)
    gidx1 = (gidx[e0:] - k1 * n).reshape(e1 // _BB, _BB)
    omap1 = omap_f[e0:].reshape(e1 // _BB, _BB)

    bm = 2000
    y0 = _matmul(x, kernel[:k0], bm)
    y1 = _matmul(x, kernel[k1:], bm)
    p0 = _make_sc_scatter(n_acc, cout, _BPT0)(y0, gidx0, omap0)
    p1 = _make_sc_scatter(n_acc, cout, _BPT1)(y1, gidx1, omap1)

    br = 1000
    out = pl.pallas_call(
        _combine2_body,
        grid=(n // br,),
        in_specs=[
            pl.BlockSpec((_NC, br, cout), lambda i: (0, i, 0)),
            pl.BlockSpec((_NC, br, cout), lambda i: (0, i, 0)),
            pl.BlockSpec((1, cout), lambda i: (0, 0)),
        ],
        out_specs=pl.BlockSpec((br, cout), lambda i: (i, 0)),
        out_shape=jax.ShapeDtypeStruct((n, cout), jnp.float32),
    )(p0, p1, bias.reshape(1, cout))
    return out


# speedup vs baseline: 1.0393x; 1.0103x over previous
import functools

import jax
import jax.numpy as jnp
from jax import lax
from jax.experimental import pallas as pl
from jax.experimental.pallas import tpu as pltpu
from jax.experimental.pallas import tpu_sc as plsc

_NC = 2
_NS = 16
_NW = _NC * _NS
_BB = 128
_BPT0 = 40
_BPT1 = 40


def _matmul_body(x_ref, w_ref, y_ref):
    y_ref[0] = jnp.dot(x_ref[...], w_ref[0], preferred_element_type=jnp.float32)


def _combine_pair_body(p_ref, b_ref, o_ref):
    o_ref[...] = p_ref[0] + p_ref[1] + b_ref[...]


def _combine_final_body(h_ref, q_ref, o_ref):
    o_ref[...] = h_ref[...] + q_ref[0] + q_ref[1]


def _make_sc_scatter(n_acc, cout, bpt):
    rpt = n_acc // _NS

    mesh = plsc.VectorSubcoreMesh(
        core_axis_name="c", subcore_axis_name="s",
        num_cores=_NC, num_subcores=_NS)

    @functools.partial(
        pl.kernel,
        out_type=jax.ShapeDtypeStruct((_NC, n_acc, cout), jnp.float32),
        mesh=mesh,
        scratch_types=[
            pltpu.VMEM((bpt, _BB), jnp.int32),
            pltpu.VMEM((bpt, _BB), jnp.int32),
            pltpu.VMEM((2, _BB, cout), jnp.float32),
            pltpu.VMEM_SHARED((n_acc, cout), jnp.float32),
            pltpu.SemaphoreType.DMA,
            pltpu.SemaphoreType.DMA,
        ],
    )
    def sc_scatter(y_hbm, gidx_hbm, omap_hbm, out_hbm,
                   gidx_v, omap_v, rows_v, acc_sh, gs0, gs1):
        cid = lax.axis_index("c")
        sid = lax.axis_index("s")
        wid = cid * _NS + sid

        gsems = (gs0, gs1)

        def gather(j, b):
            return pltpu.async_copy(y_hbm.at[gidx_v.at[j]], rows_v.at[b],
                                    gsems[b])

        def gather_wait(j, b):
            pltpu.make_async_copy(y_hbm.at[gidx_v.at[j]], rows_v.at[b],
                                  gsems[b]).wait()

        def scatter_sync(j, b):
            pltpu.sync_copy(rows_v.at[b], acc_sh.at[omap_v.at[j]], add=True)

        pltpu.sync_copy(gidx_hbm.at[pl.ds(wid * bpt, bpt)], gidx_v)
        pltpu.sync_copy(omap_hbm.at[pl.ds(wid * bpt, bpt)], omap_v)
        gather(1, 1)

        zero = jnp.zeros((16,), jnp.float32)

        @pl.loop(0, _BB)
        def _zero_rows(r):
            for q in range(cout // 16):
                rows_v[0, r, pl.ds(q * 16, 16)] = zero

        for t in range(rpt // _BB):
            pltpu.sync_copy(rows_v.at[0],
                            acc_sh.at[pl.ds(sid * rpt + t * _BB, _BB)])
        gather(0, 0)
        plsc.subcore_barrier()

        @pl.loop(0, bpt // 2)
        def _pairs(jj):
            j0 = jj * 2
            j1 = j0 + 1
            gather_wait(j0, 0)
            scatter_sync(j0, 0)

            @pl.when(jj < bpt // 2 - 1)
            def _():
                gather(j0 + 2, 0)

            gather_wait(j1, 1)
            scatter_sync(j1, 1)

            @pl.when(jj < bpt // 2 - 1)
            def _():
                gather(j1 + 2, 1)

        plsc.subcore_barrier()
        pltpu.sync_copy(acc_sh.at[pl.ds(sid * rpt, rpt)],
                        out_hbm.at[cid, pl.ds(sid * rpt, rpt)])

    return sc_scatter


def _matmul(x, w, bm):
    n, cin = x.shape
    nk, _, cout = w.shape
    y = pl.pallas_call(
        _matmul_body,
        grid=(n // bm, nk),
        in_specs=[
            pl.BlockSpec((bm, cin), lambda i, k: (i, 0)),
            pl.BlockSpec((1, cin, cout), lambda i, k: (k, 0, 0)),
        ],
        out_specs=pl.BlockSpec((1, bm, cout), lambda i, k: (k, i, 0)),
        out_shape=jax.ShapeDtypeStruct((nk, n, cout), jnp.float32),
    )(x, w)
    return y.reshape(nk * n, cout)


def kernel(x, imap, omap, kernel, bias):
    n, cin = x.shape
    kvol, epk = imap.shape
    cout = kernel.shape[2]

    e_total = kvol * epk
    e0 = _NW * _BPT0 * _BB
    e1 = _NW * _BPT1 * _BB
    n_acc = ((n + _NS * _BB - 1) // (_NS * _BB)) * (_NS * _BB)

    k0 = -(-e0 // epk)
    k1 = e0 // epk

    gidx = (jnp.arange(kvol, dtype=jnp.int32)[:, None] * n + imap).ravel()
    omap_f = omap.ravel()
    pad = e0 + e1 - e_total
    pad_ids = jnp.arange(pad, dtype=jnp.int32)
    gidx = jnp.concatenate([gidx, k1 * n + (pad_ids * 53) % ((kvol - k1) * n)])
    omap_f = jnp.concatenate([omap_f, n + pad_ids % (n_acc - n)])

    gidx0 = gidx[:e0].reshape(e0 // _BB, _BB)
    omap0 = omap_f[:e0].reshape(e0 // _BB, _BB)
    gidx1 = (gidx[e0:] - k1 * n).reshape(e1 // _BB, _BB)
    omap1 = omap_f[e0:].reshape(e1 // _BB, _BB)

    bm = 2000
    y0 = _matmul(x, kernel[:k0], bm)
    y1 = _matmul(x, kernel[k1:], bm)
    p0 = _make_sc_scatter(n_acc, cout, _BPT0)(y0, gidx0, omap0)
    p1 = _make_sc_scatter(n_acc, cout, _BPT1)(y1, gidx1, omap1)

    br = 1000
    half = pl.pallas_call(
        _combine_pair_body,
        grid=(n // br,),
        in_specs=[
            pl.BlockSpec((_NC, br, cout), lambda i: (0, i, 0)),
            pl.BlockSpec((1, cout), lambda i: (0, 0)),
        ],
        out_specs=pl.BlockSpec((br, cout), lambda i: (i, 0)),
        out_shape=jax.ShapeDtypeStruct((n, cout), jnp.float32),
    )(p0, bias.reshape(1, cout))
    out = pl.pallas_call(
        _combine_final_body,
        grid=(n // br,),
        in_specs=[
            pl.BlockSpec((br, cout), lambda i: (i, 0)),
            pl.BlockSpec((_NC, br, cout), lambda i: (0, i, 0)),
        ],
        out_specs=pl.BlockSpec((br, cout), lambda i: (i, 0)),
        out_shape=jax.ShapeDtypeStruct((n, cout), jnp.float32),
    )(half, p1)
    return out
